# baseline (speedup 1.0000x reference)
import jax
import jax.numpy as jnp
from jax import lax
from jax.experimental import pallas as pl
from jax.experimental.pallas import tpu as pltpu
from jax.experimental.pallas import tpu_sc as plsc

N_NODES = 10000
N_HE = 10000
E = 320000

NC = 2
NS = 16
NW = NC * NS

S_ACC = 10240
ROWS_PER_TILE = S_ACC // NS
CH = 128
K = 8
EDGES_PER_W = 10240
EP = NW * EDGES_PER_W
NCHUNK = EDGES_PER_W // CH
NOUT = NCHUNK // K
CPB = 16
KI = 8
NOI = CPB // KI
NCH_F = 144
NCH_S = 16
NB_F = NCH_F // CPB
NB_S = NCH_S // CPB
EF = NS * NCH_F * CH
ES = NS * NCH_S * CH

_f32 = jnp.float32
CNTW = 8


def _mesh():
    return plsc.VectorSubcoreMesh(core_axis_name="c", subcore_axis_name="s")


def _count_body(ones_hbm, zeros_hbm, dsts, out, idx_d, ones_v, accum):
    cid = lax.axis_index("c")
    sid = lax.axis_index("s")
    wid = sid * NC + cid
    row0 = sid * ROWS_PER_TILE
    pltpu.sync_copy(zeros_hbm, accum.at[pl.ds(row0, ROWS_PER_TILE)])
    pltpu.sync_copy(ones_hbm, ones_v)
    pltpu.sync_copy(dsts.at[wid], idx_d)
    plsc.subcore_barrier()

    def outer(j0, carry):
        for j in range(K):
            pltpu.sync_copy(ones_v, accum.at[idx_d.at[j0 * K + j]], add=True)
        return carry

    lax.fori_loop(0, NOUT, outer, 0)
    plsc.subcore_barrier()
    sl = pl.ds(row0, ROWS_PER_TILE)
    pltpu.sync_copy(accum.at[sl], out.at[cid, sl])


def _counts(ones_hbm, zeros_hbm, dsts):
    kern = pl.kernel(
        _count_body,
        out_type=jax.ShapeDtypeStruct((NC, S_ACC, CNTW), _f32),
        mesh=_mesh(),
        compiler_params=pltpu.CompilerParams(use_tc_tiling_on_sc=False),
        scratch_types=[
            pltpu.VMEM((NCHUNK, CH), jnp.int32),
            pltpu.VMEM((CH, CNTW), _f32),
            pltpu.VMEM_SHARED((S_ACC, CNTW), _f32),
        ],
    )
    return kern(ones_hbm, zeros_hbm, dsts)


def _pass_factory(nb_f, nb_s):
  def _pass_body(table, zeros_hbm, srcs_f, dsts_f, srcs_s, dsts_s, out,
               idx_s, idx_d, rows0, rows1, accum, gsem):
    cid = lax.axis_index("c")
    sid = lax.axis_index("s")
    row0 = sid * ROWS_PER_TILE
    pltpu.sync_copy(zeros_hbm, accum.at[pl.ds(row0, ROWS_PER_TILE)])
    plsc.subcore_barrier()

    rows = (rows0, rows1)

    def run(srcs, dsts, nb):
        def batch(b, carry):
            pltpu.sync_copy(srcs.at[sid, pl.ds(b * CPB, CPB)], idx_s)
            pltpu.sync_copy(dsts.at[sid, pl.ds(b * CPB, CPB)], idx_d)
            pltpu.async_copy(table.at[idx_s.at[0]], rows0, gsem)

            def mid(m, c2):
                for j in range(KI):
                    jj = m * KI + j
                    cur = rows[j % 2]
                    nxt = rows[(j + 1) % 2]
                    pltpu.make_async_copy(table.at[idx_s.at[0]], cur,
                                          gsem).wait()
                    if j < KI - 1:
                        pltpu.async_copy(table.at[idx_s.at[jj + 1]], nxt, gsem)
                    else:
                        @pl.when(m < NOI - 1)
                        def _():
                            pltpu.async_copy(table.at[idx_s.at[jj + 1]], nxt,
                                             gsem)
                    pltpu.sync_copy(cur, accum.at[idx_d.at[jj]], add=True)
                return c2

            lax.fori_loop(0, NOI, mid, 0)
            return carry

        lax.fori_loop(0, nb, batch, 0)

    @pl.when(cid == 0)
    def _():
        run(srcs_f, dsts_f, nb_f)

    if nb_s:
        @pl.when(cid == 1)
        def _():
            run(srcs_s, dsts_s, nb_s)

    plsc.subcore_barrier()
    sl = pl.ds(row0, ROWS_PER_TILE)
    pltpu.sync_copy(accum.at[sl], out.at[cid, sl])

  return _pass_body


def _segsum(table, zeros_hbm, srcs_f, dsts_f, srcs_s, dsts_s, feat,
            nb_f=NB_F, nb_s=NB_S):
    params = None
    if feat < 128:
        params = pltpu.CompilerParams(use_tc_tiling_on_sc=False)
    kern = pl.kernel(
        _pass_factory(nb_f, nb_s),
        out_type=jax.ShapeDtypeStruct((NC, S_ACC, feat), _f32),
        mesh=_mesh(),
        compiler_params=params,
        scratch_types=[
            pltpu.VMEM((CPB, CH), jnp.int32),
            pltpu.VMEM((CPB, CH), jnp.int32),
            pltpu.VMEM((CH, feat), _f32),
            pltpu.VMEM((CH, feat), _f32),
            pltpu.VMEM_SHARED((S_ACC, feat), _f32),
            pltpu.SemaphoreType.DMA,
        ],
    )
    return kern(table, zeros_hbm, srcs_f, dsts_f, srcs_s, dsts_s)


_BM = 512


def _matmul_kernel(x_ref, w_ref, o_ref):
    o_ref[...] = jnp.dot(x_ref[...], w_ref[...], preferred_element_type=_f32)


def _matmul(x, w):
    m, kdim = x.shape
    n = w.shape[1]
    return pl.pallas_call(
        _matmul_kernel,
        grid=(m // _BM,),
        in_specs=[pl.BlockSpec((_BM, kdim), lambda i: (i, 0)),
                  pl.BlockSpec((kdim, n), lambda i: (0, 0))],
        out_specs=pl.BlockSpec((_BM, n), lambda i: (i, 0)),
        out_shape=jax.ShapeDtypeStruct((m, n), _f32),
    )(x, w)


def _mid_kernel(p_ref, cnt_ref, o_ref):
    p = p_ref[...]
    c = cnt_ref[...]
    cnt = c[0, :, :1] + c[1, :, :1]
    inv = jnp.where(cnt > 0, 1.0 / cnt, 0.0)
    o_ref[...] = (p[0] + p[1]) * inv


def _mid(partials, cnt):
    feat = partials.shape[-1]
    return pl.pallas_call(
        _mid_kernel,
        grid=(S_ACC // _BM,),
        in_specs=[pl.BlockSpec((2, _BM, feat), lambda i: (0, i, 0)),
                  pl.BlockSpec((2, _BM, CNTW), lambda i: (0, i, 0))],
        out_specs=pl.BlockSpec((_BM, feat), lambda i: (i, 0)),
        out_shape=jax.ShapeDtypeStruct((S_ACC, feat), _f32),
    )(partials, cnt)


def _post_kernel(p_ref, cnt_ref, b_ref, w_ref, o_ref):
    p = p_ref[...]
    c = cnt_ref[...]
    cnt = c[0, :, :1] + c[1, :, :1]
    inv = jnp.where(cnt > 0, 1.0 / cnt, 0.0)
    h = jnp.maximum((p[0] + p[1]) * inv + b_ref[...], 0.0)
    o_ref[...] = jnp.dot(h, w_ref[...], preferred_element_type=_f32)


def _post_matmul(partials, cnt, b, w):
    feat = partials.shape[-1]
    n = w.shape[1]
    return pl.pallas_call(
        _post_kernel,
        grid=(S_ACC // _BM,),
        in_specs=[pl.BlockSpec((2, _BM, feat), lambda i: (0, i, 0)),
                  pl.BlockSpec((2, _BM, CNTW), lambda i: (0, i, 0)),
                  pl.BlockSpec((1, feat), lambda i: (0, 0)),
                  pl.BlockSpec((feat, n), lambda i: (0, 0))],
        out_specs=pl.BlockSpec((_BM, n), lambda i: (i, 0)),
        out_shape=jax.ShapeDtypeStruct((S_ACC, n), _f32),
    )(partials, cnt, b, w)


def _final_kernel(p_ref, cnt_ref, b_ref, o_ref):
    p = p_ref[...]
    c = cnt_ref[...]
    cnt = c[0, :, :1] + c[1, :, :1]
    inv = jnp.where(cnt > 0, 1.0 / cnt, 0.0)
    o_ref[...] = jnp.maximum((p[0] + p[1]) * inv + b_ref[...], 0.0)


def _final(partials, cnt, b):
    feat = partials.shape[-1]
    return pl.pallas_call(
        _final_kernel,
        grid=(S_ACC // _BM,),
        in_specs=[pl.BlockSpec((2, _BM, feat), lambda i: (0, i, 0)),
                  pl.BlockSpec((2, _BM, CNTW), lambda i: (0, i, 0)),
                  pl.BlockSpec((1, feat), lambda i: (0, 0))],
        out_specs=pl.BlockSpec((_BM, feat), lambda i: (i, 0)),
        out_shape=jax.ShapeDtypeStruct((S_ACC, feat), _f32),
    )(partials, cnt, b)


def kernel(x, edge, W1, b1, W2, b2, W3, b3):
    node = edge[0].astype(jnp.int32)
    he = edge[1].astype(jnp.int32)
    pad = jnp.full((EP - E,), S_ACC - 1, jnp.int32)
    node_flat = jnp.concatenate([node, pad])
    he_flat = jnp.concatenate([he, pad])
    node_p = node_flat.reshape(NW, NCHUNK, CH)
    he_p = he_flat.reshape(NW, NCHUNK, CH)
    node_f = node_flat[:EF].reshape(NS, NCH_F, CH)
    node_s = node_flat[EF:].reshape(NS, NCH_S, CH)
    he_f = he_flat[:EF].reshape(NS, NCH_F, CH)
    he_s = he_flat[EF:].reshape(NS, NCH_S, CH)
    node_a = node_flat.reshape(NS, NCHUNK * NC, CH)
    he_a = he_flat.reshape(NS, NCHUNK * NC, CH)
    xp = jnp.pad(x, ((0, S_ACC - N_NODES), (0, 0)))

    ones_c = jnp.ones((CH, CNTW), _f32)
    z_c = jnp.zeros((ROWS_PER_TILE, CNTW), _f32)
    zeros = {f: jnp.zeros((ROWS_PER_TILE, f), _f32) for f in (128, 64, 32)}

    cntn = _counts(ones_c, z_c, node_p)
    cnth = _counts(ones_c, z_c, he_p)

    t = _matmul(xp, W1)
    ws = [W2, W3, None]
    bs = [b1, b2, b3]
    for w_next, b in zip(ws, bs):
        feat = t.shape[-1]
        if feat == 128:
            p1 = _segsum(t, zeros[feat], node_a, he_a, node_s, he_s, feat,
                         nb_f=NCHUNK * NC // CPB, nb_s=0)
            m = _mid(p1, cnth)
            p2 = _segsum(m, zeros[feat], he_a, node_a, he_s, node_s, feat,
                         nb_f=NCHUNK * NC // CPB, nb_s=0)
        else:
            p1 = _segsum(t, zeros[feat], node_f, he_f, node_s, he_s, feat)
            m = _mid(p1, cnth)
            p2 = _segsum(m, zeros[feat], he_f, node_f, he_s, node_s, feat)
        b2d = b.reshape(1, feat)
        if w_next is None:
            t = _final(p2, cntn, b2d)
        else:
            t = _post_matmul(p2, cntn, b2d, w_next)
    return t[:N_NODES]


# speedup vs baseline: 1.2994x; 1.2994x over previous
import jax
import jax.numpy as jnp
from jax import lax
from jax.experimental import pallas as pl
from jax.experimental.pallas import tpu as pltpu
from jax.experimental.pallas import tpu_sc as plsc

N_NODES = 10000
N_HE = 10000
E = 320000

NC = 2
NS = 16
NW = NC * NS

S_ACC = 10240
ROWS_PER_TILE = S_ACC // NS
CH = 128
K = 8
EDGES_PER_W = 10240
EP = NW * EDGES_PER_W
NCHUNK = EDGES_PER_W // CH
NOUT = NCHUNK // K
CPB = 16
KI = 8
NOI = CPB // KI
NCH_F = 144
NCH_S = 16
NB_F = NCH_F // CPB
NB_S = NCH_S // CPB
EF = NS * NCH_F * CH
ES = NS * NCH_S * CH

_f32 = jnp.float32
CNTW = 8


def _mesh():
    return plsc.VectorSubcoreMesh(core_axis_name="c", subcore_axis_name="s")


def _count_body(ones_hbm, zeros_hbm, dsts, out, idx_d, ones_v, accum):
    cid = lax.axis_index("c")
    sid = lax.axis_index("s")
    wid = sid * NC + cid
    row0 = sid * ROWS_PER_TILE
    pltpu.sync_copy(zeros_hbm, accum.at[pl.ds(row0, ROWS_PER_TILE)])
    pltpu.sync_copy(ones_hbm, ones_v)
    pltpu.sync_copy(dsts.at[wid], idx_d)
    plsc.subcore_barrier()

    def outer(j0, carry):
        for j in range(K):
            pltpu.sync_copy(ones_v, accum.at[idx_d.at[j0 * K + j]], add=True)
        return carry

    lax.fori_loop(0, NOUT, outer, 0)
    plsc.subcore_barrier()
    sl = pl.ds(row0, ROWS_PER_TILE)
    pltpu.sync_copy(accum.at[sl], out.at[cid, sl])


def _counts(ones_hbm, zeros_hbm, dsts):
    kern = pl.kernel(
        _count_body,
        out_type=jax.ShapeDtypeStruct((NC, S_ACC, CNTW), _f32),
        mesh=_mesh(),
        compiler_params=pltpu.CompilerParams(use_tc_tiling_on_sc=False),
        scratch_types=[
            pltpu.VMEM((NCHUNK, CH), jnp.int32),
            pltpu.VMEM((CH, CNTW), _f32),
            pltpu.VMEM_SHARED((S_ACC, CNTW), _f32),
        ],
    )
    return kern(ones_hbm, zeros_hbm, dsts)


def _pass_factory(nb_f, nb_s):
  def _pass_body(table, zeros_hbm, srcs_f, dsts_f, srcs_s, dsts_s, out,
               idx_s, idx_d, rows0, rows1, accum, gsem):
    cid = lax.axis_index("c")
    sid = lax.axis_index("s")
    row0 = sid * ROWS_PER_TILE
    pltpu.sync_copy(zeros_hbm, accum.at[pl.ds(row0, ROWS_PER_TILE)])
    plsc.subcore_barrier()

    rows = (rows0, rows1)

    def run(srcs, dsts, nb):
        def batch(b, carry):
            pltpu.sync_copy(srcs.at[sid, pl.ds(b * CPB, CPB)], idx_s)
            pltpu.sync_copy(dsts.at[sid, pl.ds(b * CPB, CPB)], idx_d)
            pltpu.async_copy(table.at[idx_s.at[0]], rows0, gsem)

            def mid(m, c2):
                for j in range(KI):
                    jj = m * KI + j
                    cur = rows[j % 2]
                    nxt = rows[(j + 1) % 2]
                    pltpu.make_async_copy(table.at[idx_s.at[0]], cur,
                                          gsem).wait()
                    if j < KI - 1:
                        pltpu.async_copy(table.at[idx_s.at[jj + 1]], nxt, gsem)
                    else:
                        @pl.when(m < NOI - 1)
                        def _():
                            pltpu.async_copy(table.at[idx_s.at[jj + 1]], nxt,
                                             gsem)
                    pltpu.sync_copy(cur, accum.at[idx_d.at[jj]], add=True)
                return c2

            lax.fori_loop(0, NOI, mid, 0)
            return carry

        lax.fori_loop(0, nb, batch, 0)

    @pl.when(cid == 0)
    def _():
        run(srcs_f, dsts_f, nb_f)

    if nb_s:
        @pl.when(cid == 1)
        def _():
            run(srcs_s, dsts_s, nb_s)

    plsc.subcore_barrier()
    sl = pl.ds(row0, ROWS_PER_TILE)
    pltpu.sync_copy(accum.at[sl], out.at[cid, sl])

  return _pass_body


def _segsum(table, zeros_hbm, srcs_f, dsts_f, srcs_s, dsts_s, feat,
            nb_f=NB_F, nb_s=NB_S):
    params = None
    if feat < 128:
        params = pltpu.CompilerParams(use_tc_tiling_on_sc=False)
    kern = pl.kernel(
        _pass_factory(nb_f, nb_s),
        out_type=jax.ShapeDtypeStruct((NC, S_ACC, feat), _f32),
        mesh=_mesh(),
        compiler_params=params,
        scratch_types=[
            pltpu.VMEM((CPB, CH), jnp.int32),
            pltpu.VMEM((CPB, CH), jnp.int32),
            pltpu.VMEM((CH, feat), _f32),
            pltpu.VMEM((CH, feat), _f32),
            pltpu.VMEM_SHARED((S_ACC, feat), _f32),
            pltpu.SemaphoreType.DMA,
        ],
    )
    return kern(table, zeros_hbm, srcs_f, dsts_f, srcs_s, dsts_s)


_BM = 512


def _matmul_kernel(x_ref, w_ref, o_ref):
    o_ref[...] = jnp.dot(x_ref[...], w_ref[...], preferred_element_type=_f32)


def _matmul(x, w):
    m, kdim = x.shape
    n = w.shape[1]
    return pl.pallas_call(
        _matmul_kernel,
        grid=(m // _BM,),
        in_specs=[pl.BlockSpec((_BM, kdim), lambda i: (i, 0)),
                  pl.BlockSpec((kdim, n), lambda i: (0, 0))],
        out_specs=pl.BlockSpec((_BM, n), lambda i: (i, 0)),
        out_shape=jax.ShapeDtypeStruct((m, n), _f32),
    )(x, w)


def _mid_kernel(p_ref, cnt_ref, o_ref):
    p = p_ref[...]
    c = cnt_ref[...]
    cnt = c[0, :, :1] + c[1, :, :1]
    inv = jnp.where(cnt > 0, 1.0 / cnt, 0.0)
    o_ref[...] = (p[0] + p[1]) * inv


def _mid(partials, cnt):
    feat = partials.shape[-1]
    return pl.pallas_call(
        _mid_kernel,
        grid=(S_ACC // _BM,),
        in_specs=[pl.BlockSpec((2, _BM, feat), lambda i: (0, i, 0)),
                  pl.BlockSpec((2, _BM, CNTW), lambda i: (0, i, 0))],
        out_specs=pl.BlockSpec((_BM, feat), lambda i: (i, 0)),
        out_shape=jax.ShapeDtypeStruct((S_ACC, feat), _f32),
    )(partials, cnt)


def _post_kernel(p_ref, cnt_ref, b_ref, w_ref, o_ref):
    p = p_ref[...]
    c = cnt_ref[...]
    cnt = c[0, :, :1] + c[1, :, :1]
    inv = jnp.where(cnt > 0, 1.0 / cnt, 0.0)
    h = jnp.maximum((p[0] + p[1]) * inv + b_ref[...], 0.0)
    o_ref[...] = jnp.dot(h, w_ref[...], preferred_element_type=_f32)


def _post_matmul(partials, cnt, b, w):
    feat = partials.shape[-1]
    n = w.shape[1]
    return pl.pallas_call(
        _post_kernel,
        grid=(S_ACC // _BM,),
        in_specs=[pl.BlockSpec((2, _BM, feat), lambda i: (0, i, 0)),
                  pl.BlockSpec((2, _BM, CNTW), lambda i: (0, i, 0)),
                  pl.BlockSpec((1, feat), lambda i: (0, 0)),
                  pl.BlockSpec((feat, n), lambda i: (0, 0))],
        out_specs=pl.BlockSpec((_BM, n), lambda i: (i, 0)),
        out_shape=jax.ShapeDtypeStruct((S_ACC, n), _f32),
    )(partials, cnt, b, w)


def _final_kernel(p_ref, cnt_ref, b_ref, o_ref):
    p = p_ref[...]
    c = cnt_ref[...]
    cnt = c[0, :, :1] + c[1, :, :1]
    inv = jnp.where(cnt > 0, 1.0 / cnt, 0.0)
    o_ref[...] = jnp.maximum((p[0] + p[1]) * inv + b_ref[...], 0.0)


def _final(partials, cnt, b):
    feat = partials.shape[-1]
    return pl.pallas_call(
        _final_kernel,
        grid=(S_ACC // _BM,),
        in_specs=[pl.BlockSpec((2, _BM, feat), lambda i: (0, i, 0)),
                  pl.BlockSpec((2, _BM, CNTW), lambda i: (0, i, 0)),
                  pl.BlockSpec((1, feat), lambda i: (0, 0))],
        out_specs=pl.BlockSpec((_BM, feat), lambda i: (i, 0)),
        out_shape=jax.ShapeDtypeStruct((S_ACC, feat), _f32),
    )(partials, cnt, b)


def kernel(x, edge, W1, b1, W2, b2, W3, b3):
    node = edge[0].astype(jnp.int32)
    he = edge[1].astype(jnp.int32)
    pad = jnp.full((EP - E,), S_ACC - 1, jnp.int32)
    node_flat = jnp.concatenate([node, pad])
    he_flat = jnp.concatenate([he, pad])
    node_p = node_flat.reshape(NW, NCHUNK, CH)
    he_p = he_flat.reshape(NW, NCHUNK, CH)
    node_f = node_flat[:EF].reshape(NS, NCH_F, CH)
    node_s = node_flat[EF:].reshape(NS, NCH_S, CH)
    he_f = he_flat[:EF].reshape(NS, NCH_F, CH)
    he_s = he_flat[EF:].reshape(NS, NCH_S, CH)
    node_a = node_flat.reshape(NS, NCHUNK * NC, CH)
    he_a = he_flat.reshape(NS, NCHUNK * NC, CH)
    xp = jnp.pad(x, ((0, S_ACC - N_NODES), (0, 0)))

    ones_c = jnp.ones((CH, CNTW), _f32)
    z_c = jnp.zeros((ROWS_PER_TILE, CNTW), _f32)
    zeros = {f: jnp.zeros((ROWS_PER_TILE, f), _f32) for f in (128, 64, 32)}

    cntn = _counts(ones_c, z_c, node_p)
    cnth = _counts(ones_c, z_c, he_p)

    t = _matmul(xp, W1)
    ws = [W2, W3, None]
    bs = [b1, b2, b3]
    for w_next, b in zip(ws, bs):
        feat = t.shape[-1]
        p1 = _segsum(t, zeros[feat], node_f, he_f, node_s, he_s, feat)
        m = _mid(p1, cnth)
        p2 = _segsum(m, zeros[feat], he_f, node_f, he_s, node_s, feat)
        b2d = b.reshape(1, feat)
        if w_next is None:
            t = _final(p2, cntn, b2d)
        else:
            t = _post_matmul(p2, cntn, b2d, w_next)
    return t[:N_NODES]


# speedup vs baseline: 1.7153x; 1.3201x over previous
import jax
import jax.numpy as jnp
from jax import lax
from jax.experimental import pallas as pl
from jax.experimental.pallas import tpu as pltpu
from jax.experimental.pallas import tpu_sc as plsc

N_NODES = 10000
N_HE = 10000
E = 320000

NC = 2
NS = 16
NW = NC * NS

S_ACC = 10240
ROWS_PER_TILE = S_ACC // NS
CH = 128
K = 8
EDGES_PER_W = 10240
EP = NW * EDGES_PER_W
NCHUNK = EDGES_PER_W // CH
NOUT = NCHUNK // K
CPB = 16
KI = 8
NOI = CPB // KI
NCH_F = 144
NCH_S = 16
NB_F = NCH_F // CPB
NB_S = NCH_S // CPB
EF = NS * NCH_F * CH
ES = NS * NCH_S * CH

_f32 = jnp.float32
CNTW = 8


def _mesh():
    return plsc.VectorSubcoreMesh(core_axis_name="c", subcore_axis_name="s")


def _count_body(ones_hbm, zeros_hbm, dsts, out, idx_d, ones_v, accum):
    cid = lax.axis_index("c")
    sid = lax.axis_index("s")
    wid = sid * NC + cid
    row0 = sid * ROWS_PER_TILE
    pltpu.sync_copy(zeros_hbm, accum.at[pl.ds(row0, ROWS_PER_TILE)])
    pltpu.sync_copy(ones_hbm, ones_v)
    pltpu.sync_copy(dsts.at[wid], idx_d)
    plsc.subcore_barrier()

    def outer(j0, carry):
        for j in range(K):
            pltpu.sync_copy(ones_v, accum.at[idx_d.at[j0 * K + j]], add=True)
        return carry

    lax.fori_loop(0, NOUT, outer, 0)
    plsc.subcore_barrier()
    sl = pl.ds(row0, ROWS_PER_TILE)
    pltpu.sync_copy(accum.at[sl], out.at[cid, sl])


def _counts(ones_hbm, zeros_hbm, dsts):
    kern = pl.kernel(
        _count_body,
        out_type=jax.ShapeDtypeStruct((NC, S_ACC, CNTW), _f32),
        mesh=_mesh(),
        compiler_params=pltpu.CompilerParams(use_tc_tiling_on_sc=False),
        scratch_types=[
            pltpu.VMEM((NCHUNK, CH), jnp.int32),
            pltpu.VMEM((CH, CNTW), _f32),
            pltpu.VMEM_SHARED((S_ACC, CNTW), _f32),
        ],
    )
    return kern(ones_hbm, zeros_hbm, dsts)


def _pass_factory(nb_f, nb_s):
  def _pass_body(table, zeros_hbm, srcs_f, dsts_f, srcs_s, dsts_s, out,
               idx_s, idx_d, rows0, rows1, accum, gsem):
    cid = lax.axis_index("c")
    sid = lax.axis_index("s")
    row0 = sid * ROWS_PER_TILE
    pltpu.sync_copy(zeros_hbm, accum.at[pl.ds(row0, ROWS_PER_TILE)])
    plsc.subcore_barrier()

    rows = (rows0, rows1)

    def run(srcs, dsts, nb):
        def batch(b, carry):
            pltpu.sync_copy(srcs.at[sid, pl.ds(b * CPB, CPB)], idx_s)
            pltpu.sync_copy(dsts.at[sid, pl.ds(b * CPB, CPB)], idx_d)
            pltpu.async_copy(table.at[idx_s.at[0]], rows0, gsem)

            def mid(m, c2):
                for j in range(KI):
                    jj = m * KI + j
                    cur = rows[j % 2]
                    nxt = rows[(j + 1) % 2]
                    pltpu.make_async_copy(table.at[idx_s.at[0]], cur,
                                          gsem).wait()
                    if j < KI - 1:
                        pltpu.async_copy(table.at[idx_s.at[jj + 1]], nxt, gsem)
                    else:
                        @pl.when(m < NOI - 1)
                        def _():
                            pltpu.async_copy(table.at[idx_s.at[jj + 1]], nxt,
                                             gsem)
                    pltpu.sync_copy(cur, accum.at[idx_d.at[jj]], add=True)
                return c2

            lax.fori_loop(0, NOI, mid, 0)
            return carry

        lax.fori_loop(0, nb, batch, 0)

    @pl.when(cid == 0)
    def _():
        run(srcs_f, dsts_f, nb_f)

    if nb_s:
        @pl.when(cid == 1)
        def _():
            run(srcs_s, dsts_s, nb_s)

    plsc.subcore_barrier()
    sl = pl.ds(row0, ROWS_PER_TILE)
    pltpu.sync_copy(accum.at[sl], out.at[cid, sl])

  return _pass_body


def _spmem_pass_body(table, zeros_hbm, srcs, dsts, out,
                     idx_s, idx_d, rows0, tshared, accum):
    cid = lax.axis_index("c")
    sid = lax.axis_index("s")
    row0 = sid * ROWS_PER_TILE
    sl = pl.ds(row0, ROWS_PER_TILE)
    pltpu.sync_copy(zeros_hbm, accum.at[sl])
    pltpu.sync_copy(table.at[sl], tshared.at[sl])
    plsc.subcore_barrier()

    def batch(b, carry):
        pltpu.sync_copy(srcs.at[cid, sid, pl.ds(b * CPB, CPB)], idx_s)
        pltpu.sync_copy(dsts.at[cid, sid, pl.ds(b * CPB, CPB)], idx_d)

        def mid(m, c2):
            for j in range(KI):
                jj = m * KI + j
                pltpu.sync_copy(tshared.at[idx_s.at[jj]], rows0)
                pltpu.sync_copy(rows0, accum.at[idx_d.at[jj]], add=True)
            return c2

        lax.fori_loop(0, NOI, mid, 0)
        return carry

    lax.fori_loop(0, NCHUNK // CPB, batch, 0)
    plsc.subcore_barrier()
    pltpu.sync_copy(accum.at[sl], out.at[cid, sl])


def _segsum_spmem(table, zeros_hbm, srcs, dsts, feat):
    kern = pl.kernel(
        _spmem_pass_body,
        out_type=jax.ShapeDtypeStruct((NC, S_ACC, feat), _f32),
        mesh=_mesh(),
        compiler_params=pltpu.CompilerParams(use_tc_tiling_on_sc=False),
        scratch_types=[
            pltpu.VMEM((CPB, CH), jnp.int32),
            pltpu.VMEM((CPB, CH), jnp.int32),
            pltpu.VMEM((CH, feat), _f32),
            pltpu.VMEM_SHARED((S_ACC, feat), _f32),
            pltpu.VMEM_SHARED((S_ACC, feat), _f32),
        ],
    )
    return kern(table, zeros_hbm, srcs, dsts)


def _segsum(table, zeros_hbm, srcs_f, dsts_f, srcs_s, dsts_s, feat,
            nb_f=NB_F, nb_s=NB_S):
    params = None
    if feat < 128:
        params = pltpu.CompilerParams(use_tc_tiling_on_sc=False)
    kern = pl.kernel(
        _pass_factory(nb_f, nb_s),
        out_type=jax.ShapeDtypeStruct((NC, S_ACC, feat), _f32),
        mesh=_mesh(),
        compiler_params=params,
        scratch_types=[
            pltpu.VMEM((CPB, CH), jnp.int32),
            pltpu.VMEM((CPB, CH), jnp.int32),
            pltpu.VMEM((CH, feat), _f32),
            pltpu.VMEM((CH, feat), _f32),
            pltpu.VMEM_SHARED((S_ACC, feat), _f32),
            pltpu.SemaphoreType.DMA,
        ],
    )
    return kern(table, zeros_hbm, srcs_f, dsts_f, srcs_s, dsts_s)


_BM = 512


def _matmul_kernel(x_ref, w_ref, o_ref):
    o_ref[...] = jnp.dot(x_ref[...], w_ref[...], preferred_element_type=_f32)


def _matmul(x, w):
    m, kdim = x.shape
    n = w.shape[1]
    return pl.pallas_call(
        _matmul_kernel,
        grid=(m // _BM,),
        in_specs=[pl.BlockSpec((_BM, kdim), lambda i: (i, 0)),
                  pl.BlockSpec((kdim, n), lambda i: (0, 0))],
        out_specs=pl.BlockSpec((_BM, n), lambda i: (i, 0)),
        out_shape=jax.ShapeDtypeStruct((m, n), _f32),
    )(x, w)


def _mid_kernel(p_ref, cnt_ref, o_ref):
    p = p_ref[...]
    c = cnt_ref[...]
    cnt = c[0, :, :1] + c[1, :, :1]
    inv = jnp.where(cnt > 0, 1.0 / cnt, 0.0)
    o_ref[...] = (p[0] + p[1]) * inv


def _mid(partials, cnt):
    feat = partials.shape[-1]
    return pl.pallas_call(
        _mid_kernel,
        grid=(S_ACC // _BM,),
        in_specs=[pl.BlockSpec((2, _BM, feat), lambda i: (0, i, 0)),
                  pl.BlockSpec((2, _BM, CNTW), lambda i: (0, i, 0))],
        out_specs=pl.BlockSpec((_BM, feat), lambda i: (i, 0)),
        out_shape=jax.ShapeDtypeStruct((S_ACC, feat), _f32),
    )(partials, cnt)


def _post_kernel(p_ref, cnt_ref, b_ref, w_ref, o_ref):
    p = p_ref[...]
    c = cnt_ref[...]
    cnt = c[0, :, :1] + c[1, :, :1]
    inv = jnp.where(cnt > 0, 1.0 / cnt, 0.0)
    h = jnp.maximum((p[0] + p[1]) * inv + b_ref[...], 0.0)
    o_ref[...] = jnp.dot(h, w_ref[...], preferred_element_type=_f32)


def _post_matmul(partials, cnt, b, w):
    feat = partials.shape[-1]
    n = w.shape[1]
    return pl.pallas_call(
        _post_kernel,
        grid=(S_ACC // _BM,),
        in_specs=[pl.BlockSpec((2, _BM, feat), lambda i: (0, i, 0)),
                  pl.BlockSpec((2, _BM, CNTW), lambda i: (0, i, 0)),
                  pl.BlockSpec((1, feat), lambda i: (0, 0)),
                  pl.BlockSpec((feat, n), lambda i: (0, 0))],
        out_specs=pl.BlockSpec((_BM, n), lambda i: (i, 0)),
        out_shape=jax.ShapeDtypeStruct((S_ACC, n), _f32),
    )(partials, cnt, b, w)


def _final_kernel(p_ref, cnt_ref, b_ref, o_ref):
    p = p_ref[...]
    c = cnt_ref[...]
    cnt = c[0, :, :1] + c[1, :, :1]
    inv = jnp.where(cnt > 0, 1.0 / cnt, 0.0)
    o_ref[...] = jnp.maximum((p[0] + p[1]) * inv + b_ref[...], 0.0)


def _final(partials, cnt, b):
    feat = partials.shape[-1]
    return pl.pallas_call(
        _final_kernel,
        grid=(S_ACC // _BM,),
        in_specs=[pl.BlockSpec((2, _BM, feat), lambda i: (0, i, 0)),
                  pl.BlockSpec((2, _BM, CNTW), lambda i: (0, i, 0)),
                  pl.BlockSpec((1, feat), lambda i: (0, 0))],
        out_specs=pl.BlockSpec((_BM, feat), lambda i: (i, 0)),
        out_shape=jax.ShapeDtypeStruct((S_ACC, feat), _f32),
    )(partials, cnt, b)


def kernel(x, edge, W1, b1, W2, b2, W3, b3):
    node = edge[0].astype(jnp.int32)
    he = edge[1].astype(jnp.int32)
    pad = jnp.full((EP - E,), S_ACC - 1, jnp.int32)
    node_flat = jnp.concatenate([node, pad])
    he_flat = jnp.concatenate([he, pad])
    node_p = node_flat.reshape(NW, NCHUNK, CH)
    he_p = he_flat.reshape(NW, NCHUNK, CH)
    node_f = node_flat[:EF].reshape(NS, NCH_F, CH)
    node_s = node_flat[EF:].reshape(NS, NCH_S, CH)
    he_f = he_flat[:EF].reshape(NS, NCH_F, CH)
    he_s = he_flat[EF:].reshape(NS, NCH_S, CH)
    node_e = node_flat.reshape(NC, NS, NCHUNK, CH)
    he_e = he_flat.reshape(NC, NS, NCHUNK, CH)
    xp = jnp.pad(x, ((0, S_ACC - N_NODES), (0, 0)))

    ones_c = jnp.ones((CH, CNTW), _f32)
    z_c = jnp.zeros((ROWS_PER_TILE, CNTW), _f32)
    zeros = {f: jnp.zeros((ROWS_PER_TILE, f), _f32) for f in (128, 64, 32)}

    cntn = _counts(ones_c, z_c, node_p)
    cnth = _counts(ones_c, z_c, he_p)

    t = _matmul(xp, W1)
    ws = [W2, W3, None]
    bs = [b1, b2, b3]
    for w_next, b in zip(ws, bs):
        feat = t.shape[-1]
        if feat < 128:
            p1 = _segsum_spmem(t, zeros[feat], node_e, he_e, feat)
            m = _mid(p1, cnth)
            p2 = _segsum_spmem(m, zeros[feat], he_e, node_e, feat)
        else:
            p1 = _segsum(t, zeros[feat], node_f, he_f, node_s, he_s, feat)
            m = _mid(p1, cnth)
            p2 = _segsum(m, zeros[feat], he_f, node_f, he_s, node_s, feat)
        b2d = b.reshape(1, feat)
        if w_next is None:
            t = _final(p2, cntn, b2d)
        else:
            t = _post_matmul(p2, cntn, b2d, w_next)
    return t[:N_NODES]


# speedup vs baseline: 2.0242x; 1.1801x over previous
import jax
import jax.numpy as jnp
from jax import lax
from jax.experimental import pallas as pl
from jax.experimental.pallas import tpu as pltpu
from jax.experimental.pallas import tpu_sc as plsc

N_NODES = 10000
N_HE = 10000
E = 320000

NC = 2
NS = 16
NW = NC * NS

S_ACC = 10240
ROWS_PER_TILE = S_ACC // NS
CH = 128
K = 8
EDGES_PER_W = 10240
EP = NW * EDGES_PER_W
NCHUNK = EDGES_PER_W // CH
NOUT = NCHUNK // K
CPB = 16
KI = 8
NOI = CPB // KI
NCH_F = 144
NCH_S = 16
NB_F = NCH_F // CPB
NB_S = NCH_S // CPB
EF = NS * NCH_F * CH
ES = NS * NCH_S * CH

_f32 = jnp.float32
CNTW = 8


def _mesh():
    return plsc.VectorSubcoreMesh(core_axis_name="c", subcore_axis_name="s")


def _count_body(ones_hbm, zeros_hbm, dsts, out, idx_d, ones_v, accum):
    cid = lax.axis_index("c")
    sid = lax.axis_index("s")
    wid = sid * NC + cid
    row0 = sid * ROWS_PER_TILE
    pltpu.sync_copy(zeros_hbm, accum.at[pl.ds(row0, ROWS_PER_TILE)])
    pltpu.sync_copy(ones_hbm, ones_v)
    pltpu.sync_copy(dsts.at[wid], idx_d)
    plsc.subcore_barrier()

    def outer(j0, carry):
        for j in range(K):
            pltpu.sync_copy(ones_v, accum.at[idx_d.at[j0 * K + j]], add=True)
        return carry

    lax.fori_loop(0, NOUT, outer, 0)
    plsc.subcore_barrier()
    sl = pl.ds(row0, ROWS_PER_TILE)
    pltpu.sync_copy(accum.at[sl], out.at[cid, sl])


def _counts(ones_hbm, zeros_hbm, dsts):
    kern = pl.kernel(
        _count_body,
        out_type=jax.ShapeDtypeStruct((NC, S_ACC, CNTW), _f32),
        mesh=_mesh(),
        compiler_params=pltpu.CompilerParams(use_tc_tiling_on_sc=False),
        scratch_types=[
            pltpu.VMEM((NCHUNK, CH), jnp.int32),
            pltpu.VMEM((CH, CNTW), _f32),
            pltpu.VMEM_SHARED((S_ACC, CNTW), _f32),
        ],
    )
    return kern(ones_hbm, zeros_hbm, dsts)


def _pass_factory(nb_f, nb_s):
  def _pass_body(table, zeros_hbm, srcs_f, dsts_f, srcs_s, dsts_s, out,
               idx_s, idx_d, rows0, rows1, accum, gsem):
    cid = lax.axis_index("c")
    sid = lax.axis_index("s")
    row0 = sid * ROWS_PER_TILE
    pltpu.sync_copy(zeros_hbm, accum.at[pl.ds(row0, ROWS_PER_TILE)])
    plsc.subcore_barrier()

    rows = (rows0, rows1)

    def run(srcs, dsts, nb):
        def batch(b, carry):
            pltpu.sync_copy(srcs.at[sid, pl.ds(b * CPB, CPB)], idx_s)
            pltpu.sync_copy(dsts.at[sid, pl.ds(b * CPB, CPB)], idx_d)
            pltpu.async_copy(table.at[idx_s.at[0]], rows0, gsem)

            def mid(m, c2):
                for j in range(KI):
                    jj = m * KI + j
                    cur = rows[j % 2]
                    nxt = rows[(j + 1) % 2]
                    pltpu.make_async_copy(table.at[idx_s.at[0]], cur,
                                          gsem).wait()
                    if j < KI - 1:
                        pltpu.async_copy(table.at[idx_s.at[jj + 1]], nxt, gsem)
                    else:
                        @pl.when(m < NOI - 1)
                        def _():
                            pltpu.async_copy(table.at[idx_s.at[jj + 1]], nxt,
                                             gsem)
                    pltpu.sync_copy(cur, accum.at[idx_d.at[jj]], add=True)
                return c2

            lax.fori_loop(0, NOI, mid, 0)
            return carry

        lax.fori_loop(0, nb, batch, 0)

    @pl.when(cid == 0)
    def _():
        run(srcs_f, dsts_f, nb_f)

    if nb_s:
        @pl.when(cid == 1)
        def _():
            run(srcs_s, dsts_s, nb_s)

    plsc.subcore_barrier()
    sl = pl.ds(row0, ROWS_PER_TILE)
    pltpu.sync_copy(accum.at[sl], out.at[cid, sl])

  return _pass_body


def _spmem_pass_body(table, zeros_hbm, srcs, dsts, out,
                     idx_s, idx_d, rows0, tshared, accum):
    cid = lax.axis_index("c")
    sid = lax.axis_index("s")
    row0 = sid * ROWS_PER_TILE
    sl = pl.ds(row0, ROWS_PER_TILE)
    pltpu.sync_copy(zeros_hbm, accum.at[sl])
    pltpu.sync_copy(table.at[sl], tshared.at[sl])
    plsc.subcore_barrier()

    def batch(b, carry):
        pltpu.sync_copy(srcs.at[cid, sid, pl.ds(b * CPB, CPB)], idx_s)
        pltpu.sync_copy(dsts.at[cid, sid, pl.ds(b * CPB, CPB)], idx_d)

        def mid(m, c2):
            for j in range(KI):
                jj = m * KI + j
                pltpu.sync_copy(tshared.at[idx_s.at[jj]], rows0)
                pltpu.sync_copy(rows0, accum.at[idx_d.at[jj]], add=True)
            return c2

        lax.fori_loop(0, NOI, mid, 0)
        return carry

    lax.fori_loop(0, NCHUNK // CPB, batch, 0)
    plsc.subcore_barrier()
    pltpu.sync_copy(accum.at[sl], out.at[cid, sl])


def _segsum_spmem(table, zeros_hbm, srcs, dsts, feat):
    kern = pl.kernel(
        _spmem_pass_body,
        out_type=jax.ShapeDtypeStruct((NC, S_ACC, feat), _f32),
        mesh=_mesh(),
        compiler_params=pltpu.CompilerParams(use_tc_tiling_on_sc=False),
        scratch_types=[
            pltpu.VMEM((CPB, CH), jnp.int32),
            pltpu.VMEM((CPB, CH), jnp.int32),
            pltpu.VMEM((CH, feat), _f32),
            pltpu.VMEM_SHARED((S_ACC, feat), _f32),
            pltpu.VMEM_SHARED((S_ACC, feat), _f32),
        ],
    )
    return kern(table, zeros_hbm, srcs, dsts)


def _segsum(table, zeros_hbm, srcs_f, dsts_f, srcs_s, dsts_s, feat,
            nb_f=NB_F, nb_s=NB_S):
    params = None
    if feat < 128:
        params = pltpu.CompilerParams(use_tc_tiling_on_sc=False)
    kern = pl.kernel(
        _pass_factory(nb_f, nb_s),
        out_type=jax.ShapeDtypeStruct((NC, S_ACC, feat), _f32),
        mesh=_mesh(),
        compiler_params=params,
        scratch_types=[
            pltpu.VMEM((CPB, CH), jnp.int32),
            pltpu.VMEM((CPB, CH), jnp.int32),
            pltpu.VMEM((CH, feat), _f32),
            pltpu.VMEM((CH, feat), _f32),
            pltpu.VMEM_SHARED((S_ACC, feat), _f32),
            pltpu.SemaphoreType.DMA,
        ],
    )
    return kern(table, zeros_hbm, srcs_f, dsts_f, srcs_s, dsts_s)


_BM = 512


def _matmul_kernel(x_ref, w_ref, o_ref):
    o_ref[...] = jnp.dot(x_ref[...], w_ref[...], preferred_element_type=_f32)


def _matmul(x, w):
    m, kdim = x.shape
    n = w.shape[1]
    return pl.pallas_call(
        _matmul_kernel,
        grid=(m // _BM,),
        in_specs=[pl.BlockSpec((_BM, kdim), lambda i: (i, 0)),
                  pl.BlockSpec((kdim, n), lambda i: (0, 0))],
        out_specs=pl.BlockSpec((_BM, n), lambda i: (i, 0)),
        out_shape=jax.ShapeDtypeStruct((m, n), _f32),
    )(x, w)


def _mid_kernel(p_ref, cnt_ref, o_ref):
    p = p_ref[...]
    c = cnt_ref[...]
    cnt = c[0, :, :1] + c[1, :, :1]
    inv = jnp.where(cnt > 0, 1.0 / cnt, 0.0)
    o_ref[...] = (p[0] + p[1]) * inv


def _mid(partials, cnt):
    feat = partials.shape[-1]
    return pl.pallas_call(
        _mid_kernel,
        grid=(S_ACC // _BM,),
        in_specs=[pl.BlockSpec((2, _BM, feat), lambda i: (0, i, 0)),
                  pl.BlockSpec((2, _BM, CNTW), lambda i: (0, i, 0))],
        out_specs=pl.BlockSpec((_BM, feat), lambda i: (i, 0)),
        out_shape=jax.ShapeDtypeStruct((S_ACC, feat), _f32),
    )(partials, cnt)


def _post_kernel(p_ref, cnt_ref, b_ref, w_ref, o_ref):
    p = p_ref[...]
    c = cnt_ref[...]
    cnt = c[0, :, :1] + c[1, :, :1]
    inv = jnp.where(cnt > 0, 1.0 / cnt, 0.0)
    h = jnp.maximum((p[0] + p[1]) * inv + b_ref[...], 0.0)
    o_ref[...] = jnp.dot(h, w_ref[...], preferred_element_type=_f32)


def _post_matmul(partials, cnt, b, w):
    feat = partials.shape[-1]
    n = w.shape[1]
    return pl.pallas_call(
        _post_kernel,
        grid=(S_ACC // _BM,),
        in_specs=[pl.BlockSpec((2, _BM, feat), lambda i: (0, i, 0)),
                  pl.BlockSpec((2, _BM, CNTW), lambda i: (0, i, 0)),
                  pl.BlockSpec((1, feat), lambda i: (0, 0)),
                  pl.BlockSpec((feat, n), lambda i: (0, 0))],
        out_specs=pl.BlockSpec((_BM, n), lambda i: (i, 0)),
        out_shape=jax.ShapeDtypeStruct((S_ACC, n), _f32),
    )(partials, cnt, b, w)


def _final_kernel(p_ref, cnt_ref, b_ref, o_ref):
    p = p_ref[...]
    c = cnt_ref[...]
    cnt = c[0, :, :1] + c[1, :, :1]
    inv = jnp.where(cnt > 0, 1.0 / cnt, 0.0)
    o_ref[...] = jnp.maximum((p[0] + p[1]) * inv + b_ref[...], 0.0)


def _final(partials, cnt, b):
    feat = partials.shape[-1]
    return pl.pallas_call(
        _final_kernel,
        grid=(S_ACC // _BM,),
        in_specs=[pl.BlockSpec((2, _BM, feat), lambda i: (0, i, 0)),
                  pl.BlockSpec((2, _BM, CNTW), lambda i: (0, i, 0)),
                  pl.BlockSpec((1, feat), lambda i: (0, 0))],
        out_specs=pl.BlockSpec((_BM, feat), lambda i: (i, 0)),
        out_shape=jax.ShapeDtypeStruct((S_ACC, feat), _f32),
    )(partials, cnt, b)


def kernel(x, edge, W1, b1, W2, b2, W3, b3):
    node = edge[0].astype(jnp.int32)
    he = edge[1].astype(jnp.int32)
    pad = jnp.full((EP - E,), S_ACC - 1, jnp.int32)
    node_flat = jnp.concatenate([node, pad])
    he_flat = jnp.concatenate([he, pad])
    node_p = node_flat.reshape(NW, NCHUNK, CH)
    he_p = he_flat.reshape(NW, NCHUNK, CH)
    node_f = node_flat[:EF].reshape(NS, NCH_F, CH)
    node_s = node_flat[EF:].reshape(NS, NCH_S, CH)
    he_f = he_flat[:EF].reshape(NS, NCH_F, CH)
    he_s = he_flat[EF:].reshape(NS, NCH_S, CH)
    node_e = node_flat.reshape(NC, NS, NCHUNK, CH)
    he_e = he_flat.reshape(NC, NS, NCHUNK, CH)
    xp = jnp.pad(x, ((0, S_ACC - N_NODES), (0, 0)))

    ones_c = jnp.ones((CH, CNTW), _f32)
    z_c = jnp.zeros((ROWS_PER_TILE, CNTW), _f32)
    zeros = {f: jnp.zeros((ROWS_PER_TILE, f), _f32) for f in (128, 64, 32)}

    cntn = _counts(ones_c, z_c, node_p)
    cnth = _counts(ones_c, z_c, he_p)

    t = _matmul(xp, W1)
    ws = [W2, W3, None]
    bs = [b1, b2, b3]
    for w_next, b in zip(ws, bs):
        feat = t.shape[-1]

        def segsum(tab, srcs, dsts):
            if feat < 128:
                return _segsum_spmem(tab, zeros[feat], srcs, dsts, feat)
            halves = [
                _segsum_spmem(tab[:, h * 64:(h + 1) * 64], zeros[64],
                              srcs, dsts, 64)
                for h in range(2)
            ]
            return jnp.concatenate(halves, axis=-1)

        p1 = segsum(t, node_e, he_e)
        m = _mid(p1, cnth)
        p2 = segsum(m, he_e, node_e)
        b2d = b.reshape(1, feat)
        if w_next is None:
            t = _final(p2, cntn, b2d)
        else:
            t = _post_matmul(p2, cntn, b2d, w_next)
    return t[:N_NODES]
